# baseline (speedup 1.0000x reference)
import functools
import math

import jax
import jax.numpy as jnp
from jax import lax
from jax.experimental import pallas as pl
from jax.experimental.pallas import tpu as pltpu
from jax.experimental.pallas import tpu_sc as plsc

_EPS = 1e-5
_BLK = 1000


def _bn_from_stats(z, st_ref, row, g, e, n):
    m = st_ref[row : row + 1, :] / n
    v = st_ref[row + 1 : row + 2, :] / n - m * m
    return g * (z - m) * lax.rsqrt(v + _EPS) + e


def _stats_rows(arrs, h):
    rows = []
    for a in arrs:
        rows.append(jnp.sum(a, axis=0, keepdims=True))
        rows.append(jnp.sum(a * a, axis=0, keepdims=True))
    pad = 8 - len(rows)
    if pad:
        rows.append(jnp.zeros((pad, h), jnp.float32))
    return jnp.concatenate(rows, axis=0)


def _accum_stats(i, st_ref, st):
    @pl.when(i == 0)
    def _():
        st_ref[...] = st

    @pl.when(i > 0)
    def _():
        st_ref[...] = st_ref[...] + st


def _a1_body(feat_ref, w1_ref, b1_ref, z1_ref, st_ref):
    i = pl.program_id(0)
    z = jnp.dot(feat_ref[...], w1_ref[...], preferred_element_type=jnp.float32)
    z = z + b1_ref[...]
    z1_ref[...] = z
    _accum_stats(i, st_ref, _stats_rows([z], z.shape[1]))


def _a2_body(z1_ref, st1_ref, g1_ref, e1_ref, w2_ref, b2_ref, z2_ref, st_ref, *, n):
    i = pl.program_id(0)
    h1 = jnp.maximum(_bn_from_stats(z1_ref[...], st1_ref, 0, g1_ref[...], e1_ref[...], n), 0.0)
    z = jnp.dot(h1, w2_ref[...], preferred_element_type=jnp.float32) + b2_ref[...]
    z2_ref[...] = z
    _accum_stats(i, st_ref, _stats_rows([z], z.shape[1]))


def _a3_body(z2_ref, st2_ref, g2_ref, e2_ref, wfc_ref, bfc_ref,
             h_ref, hq_ref, coff_ref, clr_ref, *, n):
    h = jnp.maximum(_bn_from_stats(z2_ref[...], st2_ref, 0, g2_ref[...], e2_ref[...], n), 0.0)
    h_ref[...] = h
    for q in range(8):
        hq_ref[q] = h[:, q * 32 : (q + 1) * 32]
    logits = jnp.dot(h, wfc_ref[...], preferred_element_type=jnp.float32) + bfc_ref[...]
    mx = jnp.max(logits, axis=-1, keepdims=True)
    ex = jnp.exp(logits - mx)
    p = ex / jnp.sum(ex, axis=-1, keepdims=True)
    m1 = jnp.max(p, axis=-1, keepdims=True)
    ismax = p >= m1
    nmax = jnp.sum(ismax.astype(jnp.float32), axis=-1, keepdims=True)
    p2 = jnp.max(jnp.where(ismax, -1.0, p), axis=-1, keepdims=True)
    second = jnp.where(nmax > 1.5, m1, p2)
    unclear = (m1 - second) < 0.1
    coff_ref[...] = jnp.where(unclear, n, 0).astype(jnp.int32)
    clr_ref[...] = jnp.where(unclear, 0.0, 1.0)


def _c0_body(h_ref, wsa_ref, wx_ref, bx_ref, hwsa_ref, x_ref):
    h = h_ref[...]
    bb = jnp.bfloat16
    hwsa_ref[...] = jnp.dot(h, wsa_ref[...], preferred_element_type=jnp.float32)
    x_ref[...] = jnp.tanh(
        jnp.dot(h.astype(bb), wx_ref[...].astype(bb),
                preferred_element_type=jnp.float32) + bx_ref[...])


def _c1_body(hwsa_ref, smc_ref, smu_ref, cnt_ref, wna_ref, ca_ref,
             wn0_ref, c0_ref, wn1_ref, c1_ref, aall_ref, a0_ref, a1_ref, st_ref):
    i = pl.program_id(0)
    sc = jnp.concatenate([smc_ref[q] for q in range(8)], axis=-1)
    su = jnp.concatenate([smu_ref[q] for q in range(8)], axis=-1)
    inv = 1.0 / jnp.maximum(cnt_ref[...], 1.0)
    mall = (sc + su) * inv
    aall = jnp.dot(mall, wna_ref[...], preferred_element_type=jnp.float32) + ca_ref[...]
    aall = aall + hwsa_ref[...]
    aall = jnp.maximum(aall, 0.0)
    a0 = jnp.maximum(jnp.dot(sc * inv, wn0_ref[...], preferred_element_type=jnp.float32) + c0_ref[...], 0.0)
    a1 = jnp.maximum(jnp.dot(su * inv, wn1_ref[...], preferred_element_type=jnp.float32) + c1_ref[...], 0.0)
    aall_ref[...] = aall
    a0_ref[...] = a0
    a1_ref[...] = a1
    _accum_stats(i, st_ref, _stats_rows([aall, a0, a1], aall.shape[1]))


def _c2_body(aall_ref, a0_ref, a1_ref, st_ref, clr_ref, x_ref,
             ga_ref, ea_ref, g0_ref, e0_ref, gg1_ref, ee1_ref,
             wf_ref, bf_ref, wo1_ref, bo1_ref, wo2_ref, bo2_ref,
             out_ref, *, n, h_dim, nh):
    hall = _bn_from_stats(aall_ref[...], st_ref, 0, ga_ref[...], ea_ref[...], n)
    h0 = _bn_from_stats(a0_ref[...], st_ref, 2, g0_ref[...], e0_ref[...], n)
    h1 = _bn_from_stats(a1_ref[...], st_ref, 4, gg1_ref[...], ee1_ref[...], n)
    clr = clr_ref[...]
    hgp = clr * h0 + (1.0 - clr) * h1
    bb = jnp.bfloat16
    wfb = wf_ref[...].astype(bb)
    fall = jnp.tanh(jnp.dot(hall.astype(bb), wfb, preferred_element_type=jnp.float32) + bf_ref[...])
    fgp = jnp.tanh(jnp.dot(hgp.astype(bb), wfb, preferred_element_type=jnp.float32) + bf_ref[...])
    x = x_ref[...]
    scale = 1.0 / math.sqrt(float(h_dim))
    parts = []
    for hh in range(nh):
        lo = hh * h_dim
        fa = fall[:, lo : lo + h_dim]
        fg = fgp[:, lo : lo + h_dim]
        xh = x[:, lo : lo + h_dim]
        s0 = jnp.sum(fa * xh, axis=-1, keepdims=True) * scale
        s1 = jnp.sum(fg * xh, axis=-1, keepdims=True) * scale
        mx = jnp.maximum(s0, s1)
        e0 = jnp.exp(s0 - mx)
        e1 = jnp.exp(s1 - mx)
        den = e0 + e1
        parts.append((e0 / den) * fa + (e1 / den) * fg)
    agg = jnp.concatenate(parts, axis=-1)
    o = jnp.maximum(jnp.dot(agg, wo1_ref[...], preferred_element_type=jnp.float32) + bo1_ref[...], 0.0)
    out_ref[...] = jnp.dot(o, wo2_ref[...], preferred_element_type=jnp.float32) + bo2_ref[...]


_NT = 16


_FW = 32
_RING = 6


def _seg_sums_sc(hq, src2, dst2, coff, n, e_pad):
    v = n
    per_tile = e_pad // _NT
    idx_rows = per_tile // 128
    pchunks = per_tile // 2048
    acc_rows = 2 * v + 16
    cz = v // 10
    wb = (2 * v) // 10
    mesh = plsc.VectorSubcoreMesh(core_axis_name="c", subcore_axis_name="s")

    @functools.partial(
        pl.kernel,
        out_type=(
            jax.ShapeDtypeStruct((16 * v, _FW), jnp.float32),
            jax.ShapeDtypeStruct((v,), jnp.float32),
        ),
        mesh=mesh,
        scratch_types=(
            pltpu.VMEM((24, 128), jnp.int32),
            pltpu.VMEM((idx_rows, 128), jnp.int32),
            pltpu.VMEM((idx_rows, 128), jnp.int32),
            pltpu.VMEM((_RING * 128, _FW), jnp.float32),
            pltpu.VMEM((128,), jnp.float32),
            pltpu.VMEM((200,), jnp.float32),
            pltpu.VMEM_SHARED((v, _FW), jnp.float32),
            pltpu.VMEM_SHARED((acc_rows, _FW), jnp.float32),
            pltpu.VMEM_SHARED((v + 16,), jnp.float32),
            pltpu.VMEM_SHARED((v + 16,), jnp.int32),
            pltpu.SemaphoreType.DMA,
            pltpu.SemaphoreType.DMA,
            pltpu.SemaphoreType.DMA,
            pltpu.SemaphoreType.DMA,
        ),
        compiler_params=pltpu.CompilerParams(use_tc_tiling_on_sc=False),
    )
    def body(hq_h, src_h, dst_h, coff_h, out_h, cnt_h,
             ib, srca, sela, rowsb, onesb, zb, stable, acc, cacc, coffsp,
             semg, sems, semt, semn):
        c = lax.axis_index("c")
        s = lax.axis_index("s")
        lim = 2 * v + 15

        def fill16(i, _):
            onesb[pl.ds(i * 16, 16)] = jnp.ones((16,), jnp.float32)
            return 0

        lax.fori_loop(0, 8, fill16, 0)

        def zfill(i, _):
            zb[pl.ds(i * 16, 16)] = jnp.zeros((16,), jnp.float32)
            return 0

        lax.fori_loop(0, 200 // 16 + 1, zfill, 0)

        def zrows(i, _):
            for j in range(_FW // 16):
                rowsb[i, pl.ds(j * 16, 16)] = jnp.zeros((16,), jnp.float32)
            return 0

        lax.fori_loop(0, 384, zrows, 0)

        def zero_acc():
            base = s * 1248
            for j in range(4):
                pltpu.sync_copy(rowsb.at[pl.ds(0, 312)],
                                acc.at[pl.ds(base + j * 312, 312)])

            @pl.when(s == 0)
            def _():
                pltpu.sync_copy(rowsb.at[pl.ds(0, 48)],
                                acc.at[pl.ds(19968, 48)])

        def stage_table(p):
            @pl.when(s < 10)
            def _():
                pltpu.async_copy(
                    hq_h.at[pl.ds((c + 2 * p) * v + s * 1000, 1000)],
                    stable.at[pl.ds(s * 1000, 1000)], semt).wait()

        stage_table(0)

        @pl.when(s < 10)
        def _():
            pltpu.sync_copy(coff_h.at[pl.ds(s * 1000, 1000)],
                            coffsp.at[pl.ds(s * 1000, 1000)])

        zero_acc()

        @pl.when(jnp.logical_and(c == 0, s < 10))
        def _():
            for j in range(cz // 200):
                pltpu.sync_copy(zb.at[pl.ds(0, 200)],
                                cacc.at[pl.ds(s * cz + j * 200, 200)])

        plsc.subcore_barrier()

        for p in range(4):
            if p > 0:
                stage_table(p)
                lax.fori_loop(0, 384, zrows, 0)
                zero_acc()
                plsc.subcore_barrier()

            def edge_chunk(k, _, p=p):
                row0 = k * 16
                gds = {}
                sds = {}

                def buf(g):
                    return rowsb.at[pl.ds((g % _RING) * 128, 128)]

                if p == 0:
                    rb = s * idx_rows + k * 16
                    pltpu.sync_copy(src_h.at[pl.ds(rb, 16)],
                                    srca.at[pl.ds(row0, 16)])
                for g in range(_RING):
                    gds[g] = pltpu.async_copy(
                        stable.at[srca.at[row0 + g]], buf(g), semg)
                if p == 0:
                    for h2 in range(2):
                        o2 = 8 * h2
                        pltpu.sync_copy(dst_h.at[pl.ds(rb + o2, 8)],
                                        ib.at[pl.ds(8, 8)])
                        cps = [pltpu.async_copy(
                                   coffsp.at[srca.at[row0 + o2 + g]],
                                   ib.at[16 + g], semt)
                               for g in range(8)]
                        for cp in cps:
                            cp.wait()
                        for g in range(8):
                            def rloop(j, _, g=g, o2=o2):
                                sl = pl.ds(j * 16, 16)
                                dv = ib[8 + g, sl]
                                cv = ib[16 + g, sl]
                                sela[row0 + o2 + g, sl] = jnp.minimum(
                                    dv + cv, lim)
                                ib[16 + g, sl] = jnp.where(
                                    dv < v, dv,
                                    v + jnp.bitwise_and(dv, 7))
                                return 0

                            lax.fori_loop(0, 8, rloop, 0)

                        @pl.when(c == 0)
                        def _():
                            for g in range(8):
                                pltpu.async_copy(onesb,
                                                 cacc.at[ib.at[16 + g]],
                                                 semn, add=True)
                            for g in range(8):
                                pltpu.make_async_copy(onesb,
                                                      cacc.at[ib.at[16 + g]],
                                                      semn).wait()
                for g in range(16):
                    gds[g].wait()
                    sds[g] = pltpu.async_copy(
                        buf(g), acc.at[sela.at[row0 + g]], sems, add=True)
                    f = g + _RING - 2
                    if _RING <= f < 16:
                        sds[f - _RING].wait()
                        gds[f] = pltpu.async_copy(
                            stable.at[srca.at[row0 + f]], buf(f), semg)
                for g in range(16 - _RING, 16):
                    if g >= 0:
                        sds[g].wait()
                return 0

            lax.fori_loop(0, pchunks, edge_chunk, 0)
            plsc.subcore_barrier()

            if p == 0:

                @pl.when(jnp.logical_and(c == 0, s < 10))
                def _():
                    pltpu.sync_copy(cacc.at[pl.ds(s * cz, cz)],
                                    cnt_h.at[pl.ds(s * cz, cz)])

            q = c + 2 * p

            @pl.when(s < 10)
            def _():
                pltpu.sync_copy(acc.at[pl.ds(s * wb, wb)],
                                out_h.at[pl.ds(q * 2 * v + s * wb, wb)])

            plsc.subcore_barrier()

    return body(hq, src2, dst2, coff)


def kernel(feat, edge_index, W1, b1, g1, e1, W2, b2, g2, e2, Wfc, bfc,
           Wna, Wsa, ca, ga, ea, Wn0, c0, g0, e0, Wn1, c1, gg1, ee1,
           Wf, bf, Wx, bx, Wo1, bo1, Wo2, bo2):
    n, d = feat.shape
    h_dim = W1.shape[1]
    e = edge_index.shape[1]
    c_dim = Wfc.shape[1]
    nh = Wf.shape[1] // h_dim
    nblk = n // _BLK
    fn = float(n)
    r = lambda x: x.reshape(1, -1)

    row = lambda bs: pl.BlockSpec(bs, lambda i: (i, 0))
    full = lambda shape: pl.BlockSpec(shape, lambda i: (0,) * len(shape))

    z1, st1 = pl.pallas_call(
        _a1_body,
        grid=(nblk,),
        in_specs=[row((_BLK, d)), full((d, h_dim)), full((1, h_dim))],
        out_specs=[row((_BLK, h_dim)), full((8, h_dim))],
        out_shape=[jax.ShapeDtypeStruct((n, h_dim), jnp.float32),
                   jax.ShapeDtypeStruct((8, h_dim), jnp.float32)],
    )(feat, W1, r(b1))

    z2, st2 = pl.pallas_call(
        functools.partial(_a2_body, n=fn),
        grid=(nblk,),
        in_specs=[row((_BLK, h_dim)), full((8, h_dim)), full((1, h_dim)),
                  full((1, h_dim)), full((h_dim, h_dim)), full((1, h_dim))],
        out_specs=[row((_BLK, h_dim)), full((8, h_dim))],
        out_shape=[jax.ShapeDtypeStruct((n, h_dim), jnp.float32),
                   jax.ShapeDtypeStruct((8, h_dim), jnp.float32)],
    )(z1, st1, r(g1), r(e1), W2, r(b2))

    h, hq, coff, clearf = pl.pallas_call(
        functools.partial(_a3_body, n=fn),
        grid=(nblk,),
        in_specs=[row((_BLK, h_dim)), full((8, h_dim)), full((1, h_dim)),
                  full((1, h_dim)), full((h_dim, c_dim)), full((1, c_dim))],
        out_specs=[row((_BLK, h_dim)),
                   pl.BlockSpec((8, _BLK, 32), lambda i: (0, i, 0)),
                   row((_BLK, 1)), row((_BLK, 1))],
        out_shape=[jax.ShapeDtypeStruct((n, h_dim), jnp.float32),
                   jax.ShapeDtypeStruct((8, n, 32), jnp.float32),
                   jax.ShapeDtypeStruct((n, 1), jnp.int32),
                   jax.ShapeDtypeStruct((n, 1), jnp.float32)],
    )(z2, st2, r(g2), r(e2), Wfc, r(bfc))

    hwsa, xq = pl.pallas_call(
        _c0_body,
        grid=(nblk,),
        in_specs=[row((_BLK, h_dim)), full((h_dim, h_dim)),
                  full((h_dim, nh * h_dim)), full((1, nh * h_dim))],
        out_specs=[row((_BLK, h_dim)), row((_BLK, nh * h_dim))],
        out_shape=[jax.ShapeDtypeStruct((n, h_dim), jnp.float32),
                   jax.ShapeDtypeStruct((n, nh * h_dim), jnp.float32)],
    )(h, Wsa, Wx, r(bx))

    e_pad = -(-e // (_NT * 1024)) * (_NT * 1024)
    pad = e_pad - e
    src = edge_index[0]
    dst = edge_index[1]
    if pad:
        src = jnp.concatenate([src, jnp.zeros((pad,), jnp.int32)])
        dst = jnp.concatenate(
            [dst, 2 * n + (jnp.arange(pad, dtype=jnp.int32) % 8)])
    src2 = src.reshape(e_pad // 128, 128)
    dst2 = dst.reshape(e_pad // 128, 128)
    sums_flat, cnt = _seg_sums_sc(hq.reshape(8 * n, 32), src2, dst2,
                                  coff.reshape(n), n, e_pad)
    sums = sums_flat.reshape(8, -1, 32)

    qblk = lambda off: pl.BlockSpec((8, _BLK, 32), lambda i, off=off: (0, i + off, 0))
    aall, a0, a1, stc = pl.pallas_call(
        _c1_body,
        grid=(nblk,),
        in_specs=[row((_BLK, h_dim)), qblk(0), qblk(nblk), row((_BLK, 1)),
                  full((h_dim, h_dim)), full((1, h_dim)),
                  full((h_dim, h_dim)), full((1, h_dim)),
                  full((h_dim, h_dim)), full((1, h_dim))],
        out_specs=[row((_BLK, h_dim)), row((_BLK, h_dim)), row((_BLK, h_dim)),
                   full((8, h_dim))],
        out_shape=[jax.ShapeDtypeStruct((n, h_dim), jnp.float32),
                   jax.ShapeDtypeStruct((n, h_dim), jnp.float32),
                   jax.ShapeDtypeStruct((n, h_dim), jnp.float32),
                   jax.ShapeDtypeStruct((8, h_dim), jnp.float32)],
    )(hwsa, sums, sums, cnt.reshape(n, 1), Wna, r(ca), Wn0, r(c0), Wn1, r(c1))

    out = pl.pallas_call(
        functools.partial(_c2_body, n=fn, h_dim=h_dim, nh=nh),
        grid=(nblk,),
        in_specs=[row((_BLK, h_dim)), row((_BLK, h_dim)), row((_BLK, h_dim)),
                  full((8, h_dim)), row((_BLK, 1)), row((_BLK, nh * h_dim)),
                  full((1, h_dim)), full((1, h_dim)), full((1, h_dim)),
                  full((1, h_dim)), full((1, h_dim)), full((1, h_dim)),
                  full((h_dim, nh * h_dim)), full((1, nh * h_dim)),
                  full((nh * h_dim, h_dim // 2)), full((1, h_dim // 2)),
                  full((h_dim // 2, c_dim)), full((1, c_dim))],
        out_specs=[row((_BLK, c_dim))],
        out_shape=[jax.ShapeDtypeStruct((n, c_dim), jnp.float32)],
    )(aall, a0, a1, stc, clearf, xq, r(ga), r(ea), r(g0), r(e0), r(gg1), r(ee1),
      Wf, r(bf), Wo1, r(bo1), Wo2, r(bo2))[0]

    return out


# speedup vs baseline: 1.0340x; 1.0340x over previous
import functools
import math

import jax
import jax.numpy as jnp
from jax import lax
from jax.experimental import pallas as pl
from jax.experimental.pallas import tpu as pltpu
from jax.experimental.pallas import tpu_sc as plsc

_EPS = 1e-5
_BLK = 1000


def _bn_from_stats(z, st_ref, row, g, e, n):
    m = st_ref[row : row + 1, :] / n
    v = st_ref[row + 1 : row + 2, :] / n - m * m
    return g * (z - m) * lax.rsqrt(v + _EPS) + e


def _stats_rows(arrs, h):
    rows = []
    for a in arrs:
        rows.append(jnp.sum(a, axis=0, keepdims=True))
        rows.append(jnp.sum(a * a, axis=0, keepdims=True))
    pad = 8 - len(rows)
    if pad:
        rows.append(jnp.zeros((pad, h), jnp.float32))
    return jnp.concatenate(rows, axis=0)


def _accum_stats(i, st_ref, st):
    @pl.when(i == 0)
    def _():
        st_ref[...] = st

    @pl.when(i > 0)
    def _():
        st_ref[...] = st_ref[...] + st


def _a1_body(feat_ref, w1_ref, b1_ref, z1_ref, st_ref):
    i = pl.program_id(0)
    z = jnp.dot(feat_ref[...], w1_ref[...], preferred_element_type=jnp.float32)
    z = z + b1_ref[...]
    z1_ref[...] = z
    _accum_stats(i, st_ref, _stats_rows([z], z.shape[1]))


def _a2_body(z1_ref, st1_ref, g1_ref, e1_ref, w2_ref, b2_ref, z2_ref, st_ref, *, n):
    i = pl.program_id(0)
    h1 = jnp.maximum(_bn_from_stats(z1_ref[...], st1_ref, 0, g1_ref[...], e1_ref[...], n), 0.0)
    z = jnp.dot(h1, w2_ref[...], preferred_element_type=jnp.float32) + b2_ref[...]
    z2_ref[...] = z
    _accum_stats(i, st_ref, _stats_rows([z], z.shape[1]))


def _a3_body(z2_ref, st2_ref, g2_ref, e2_ref, wfc_ref, bfc_ref,
             h_ref, hq_ref, coff_ref, clr_ref, *, n):
    h = jnp.maximum(_bn_from_stats(z2_ref[...], st2_ref, 0, g2_ref[...], e2_ref[...], n), 0.0)
    h_ref[...] = h
    for q in range(8):
        hq_ref[q] = h[:, q * 32 : (q + 1) * 32]
    logits = jnp.dot(h, wfc_ref[...], preferred_element_type=jnp.float32) + bfc_ref[...]
    mx = jnp.max(logits, axis=-1, keepdims=True)
    ex = jnp.exp(logits - mx)
    p = ex / jnp.sum(ex, axis=-1, keepdims=True)
    m1 = jnp.max(p, axis=-1, keepdims=True)
    ismax = p >= m1
    nmax = jnp.sum(ismax.astype(jnp.float32), axis=-1, keepdims=True)
    p2 = jnp.max(jnp.where(ismax, -1.0, p), axis=-1, keepdims=True)
    second = jnp.where(nmax > 1.5, m1, p2)
    unclear = (m1 - second) < 0.1
    coff_ref[...] = jnp.where(unclear, n, 0).astype(jnp.int32)
    clr_ref[...] = jnp.where(unclear, 0.0, 1.0)


def _c1_body(h_ref, smc_ref, smu_ref, cnt_ref, wna_ref, wsa_ref, ca_ref,
             wn0_ref, c0_ref, wn1_ref, c1_ref, aall_ref, a0_ref, a1_ref, st_ref):
    i = pl.program_id(0)
    sc = jnp.concatenate([smc_ref[q] for q in range(8)], axis=-1)
    su = jnp.concatenate([smu_ref[q] for q in range(8)], axis=-1)
    inv = 1.0 / jnp.maximum(cnt_ref[...], 1.0)
    h = h_ref[...]
    mall = (sc + su) * inv
    aall = jnp.dot(mall, wna_ref[...], preferred_element_type=jnp.float32) + ca_ref[...]
    aall = aall + jnp.dot(h, wsa_ref[...], preferred_element_type=jnp.float32)
    aall = jnp.maximum(aall, 0.0)
    a0 = jnp.maximum(jnp.dot(sc * inv, wn0_ref[...], preferred_element_type=jnp.float32) + c0_ref[...], 0.0)
    a1 = jnp.maximum(jnp.dot(su * inv, wn1_ref[...], preferred_element_type=jnp.float32) + c1_ref[...], 0.0)
    aall_ref[...] = aall
    a0_ref[...] = a0
    a1_ref[...] = a1
    _accum_stats(i, st_ref, _stats_rows([aall, a0, a1], aall.shape[1]))


def _c2_body(aall_ref, a0_ref, a1_ref, st_ref, clr_ref, h_ref,
             ga_ref, ea_ref, g0_ref, e0_ref, gg1_ref, ee1_ref,
             wf_ref, bf_ref, wx_ref, bx_ref, wo1_ref, bo1_ref, wo2_ref, bo2_ref,
             out_ref, *, n, h_dim, nh):
    hall = _bn_from_stats(aall_ref[...], st_ref, 0, ga_ref[...], ea_ref[...], n)
    h0 = _bn_from_stats(a0_ref[...], st_ref, 2, g0_ref[...], e0_ref[...], n)
    h1 = _bn_from_stats(a1_ref[...], st_ref, 4, gg1_ref[...], ee1_ref[...], n)
    clr = clr_ref[...]
    hgp = clr * h0 + (1.0 - clr) * h1
    bb = jnp.bfloat16
    wfb = wf_ref[...].astype(bb)
    fall = jnp.tanh(jnp.dot(hall.astype(bb), wfb, preferred_element_type=jnp.float32) + bf_ref[...])
    fgp = jnp.tanh(jnp.dot(hgp.astype(bb), wfb, preferred_element_type=jnp.float32) + bf_ref[...])
    x = jnp.tanh(jnp.dot(h_ref[...].astype(bb), wx_ref[...].astype(bb), preferred_element_type=jnp.float32) + bx_ref[...])
    scale = 1.0 / math.sqrt(float(h_dim))
    parts = []
    for hh in range(nh):
        lo = hh * h_dim
        fa = fall[:, lo : lo + h_dim]
        fg = fgp[:, lo : lo + h_dim]
        xh = x[:, lo : lo + h_dim]
        s0 = jnp.sum(fa * xh, axis=-1, keepdims=True) * scale
        s1 = jnp.sum(fg * xh, axis=-1, keepdims=True) * scale
        mx = jnp.maximum(s0, s1)
        e0 = jnp.exp(s0 - mx)
        e1 = jnp.exp(s1 - mx)
        den = e0 + e1
        parts.append((e0 / den) * fa + (e1 / den) * fg)
    agg = jnp.concatenate(parts, axis=-1)
    o = jnp.maximum(jnp.dot(agg, wo1_ref[...], preferred_element_type=jnp.float32) + bo1_ref[...], 0.0)
    out_ref[...] = jnp.dot(o, wo2_ref[...], preferred_element_type=jnp.float32) + bo2_ref[...]


_NT = 16


_FW = 32
_RING = 6


def _seg_sums_sc(hq, src2, dst2, coff, n, e_pad):
    v = n
    per_tile = e_pad // _NT
    idx_rows = per_tile // 128
    pchunks = per_tile // 4096
    acc_rows = 2 * v + 16
    cz = v // 10
    wb = (2 * v) // 10
    mesh = plsc.VectorSubcoreMesh(core_axis_name="c", subcore_axis_name="s")

    @functools.partial(
        pl.kernel,
        out_type=(
            jax.ShapeDtypeStruct((16 * v, _FW), jnp.float32),
            jax.ShapeDtypeStruct((v,), jnp.float32),
        ),
        mesh=mesh,
        scratch_types=(
            pltpu.VMEM((24, 128), jnp.int32),
            pltpu.VMEM((idx_rows, 128), jnp.int32),
            pltpu.VMEM((idx_rows, 128), jnp.int32),
            pltpu.VMEM((_RING * 128, _FW), jnp.float32),
            pltpu.VMEM((128,), jnp.float32),
            pltpu.VMEM((200,), jnp.float32),
            pltpu.VMEM_SHARED((v, _FW), jnp.float32),
            pltpu.VMEM_SHARED((acc_rows, _FW), jnp.float32),
            pltpu.VMEM_SHARED((v + 16,), jnp.float32),
            pltpu.VMEM_SHARED((v + 16,), jnp.int32),
            pltpu.SemaphoreType.DMA,
            pltpu.SemaphoreType.DMA,
            pltpu.SemaphoreType.DMA,
            pltpu.SemaphoreType.DMA,
        ),
        compiler_params=pltpu.CompilerParams(use_tc_tiling_on_sc=False),
    )
    def body(hq_h, src_h, dst_h, coff_h, out_h, cnt_h,
             ib, srca, sela, rowsb, onesb, zb, stable, acc, cacc, coffsp,
             semg, sems, semt, semn):
        c = lax.axis_index("c")
        s = lax.axis_index("s")
        lim = 2 * v + 15

        def fill16(i, _):
            onesb[pl.ds(i * 16, 16)] = jnp.ones((16,), jnp.float32)
            return 0

        lax.fori_loop(0, 8, fill16, 0)

        def zfill(i, _):
            zb[pl.ds(i * 16, 16)] = jnp.zeros((16,), jnp.float32)
            return 0

        lax.fori_loop(0, 200 // 16 + 1, zfill, 0)

        def zrows(i, _):
            for j in range(_FW // 16):
                rowsb[i, pl.ds(j * 16, 16)] = jnp.zeros((16,), jnp.float32)
            return 0

        lax.fori_loop(0, 384, zrows, 0)

        def zero_acc():
            base = s * 1248
            for j in range(4):
                pltpu.sync_copy(rowsb.at[pl.ds(0, 312)],
                                acc.at[pl.ds(base + j * 312, 312)])

            @pl.when(s == 0)
            def _():
                pltpu.sync_copy(rowsb.at[pl.ds(0, 48)],
                                acc.at[pl.ds(19968, 48)])

        def stage_table(p):
            @pl.when(s < 10)
            def _():
                pltpu.async_copy(
                    hq_h.at[pl.ds((c + 2 * p) * v + s * 1000, 1000)],
                    stable.at[pl.ds(s * 1000, 1000)], semt).wait()

        stage_table(0)

        @pl.when(s < 10)
        def _():
            pltpu.sync_copy(coff_h.at[pl.ds(s * 1000, 1000)],
                            coffsp.at[pl.ds(s * 1000, 1000)])

        zero_acc()

        @pl.when(jnp.logical_and(c == 0, s < 10))
        def _():
            for j in range(cz // 200):
                pltpu.sync_copy(zb.at[pl.ds(0, 200)],
                                cacc.at[pl.ds(s * cz + j * 200, 200)])

        plsc.subcore_barrier()

        for p in range(4):
            if p > 0:
                stage_table(p)
                lax.fori_loop(0, 384, zrows, 0)
                zero_acc()
                plsc.subcore_barrier()

            def edge_chunk(k, _, p=p):
                row0 = k * 32
                gds = {}
                sds = {}

                def buf(g):
                    return rowsb.at[pl.ds((g % _RING) * 128, 128)]

                if p == 0:
                    rb = s * idx_rows + k * 32
                    pltpu.sync_copy(src_h.at[pl.ds(rb, 32)],
                                    srca.at[pl.ds(row0, 32)])
                for g in range(_RING):
                    gds[g] = pltpu.async_copy(
                        stable.at[srca.at[row0 + g]], buf(g), semg)
                if p == 0:
                    for h2 in range(4):
                        o2 = 8 * h2
                        pltpu.sync_copy(dst_h.at[pl.ds(rb + o2, 8)],
                                        ib.at[pl.ds(8, 8)])
                        cps = [pltpu.async_copy(
                                   coffsp.at[srca.at[row0 + o2 + g]],
                                   ib.at[16 + g], semt)
                               for g in range(8)]
                        for cp in cps:
                            cp.wait()
                        for g in range(8):
                            def rloop(j, _, g=g, o2=o2):
                                sl = pl.ds(j * 16, 16)
                                dv = ib[8 + g, sl]
                                cv = ib[16 + g, sl]
                                sela[row0 + o2 + g, sl] = jnp.minimum(
                                    dv + cv, lim)
                                ib[16 + g, sl] = jnp.where(
                                    dv < v, dv,
                                    v + jnp.bitwise_and(dv, 7))
                                return 0

                            lax.fori_loop(0, 8, rloop, 0)

                        @pl.when(c == 0)
                        def _():
                            for g in range(8):
                                pltpu.async_copy(onesb,
                                                 cacc.at[ib.at[16 + g]],
                                                 semn, add=True)
                            for g in range(8):
                                pltpu.make_async_copy(onesb,
                                                      cacc.at[ib.at[16 + g]],
                                                      semn).wait()
                for g in range(32):
                    gds[g].wait()
                    sds[g] = pltpu.async_copy(
                        buf(g), acc.at[sela.at[row0 + g]], sems, add=True)
                    f = g + _RING - 2
                    if _RING <= f < 32:
                        sds[f - _RING].wait()
                        gds[f] = pltpu.async_copy(
                            stable.at[srca.at[row0 + f]], buf(f), semg)
                for g in range(32 - _RING, 32):
                    if g >= 0:
                        sds[g].wait()
                return 0

            lax.fori_loop(0, pchunks, edge_chunk, 0)
            plsc.subcore_barrier()

            if p == 0:

                @pl.when(jnp.logical_and(c == 0, s < 10))
                def _():
                    pltpu.sync_copy(cacc.at[pl.ds(s * cz, cz)],
                                    cnt_h.at[pl.ds(s * cz, cz)])

            q = c + 2 * p

            @pl.when(s < 10)
            def _():
                pltpu.sync_copy(acc.at[pl.ds(s * wb, wb)],
                                out_h.at[pl.ds(q * 2 * v + s * wb, wb)])

            plsc.subcore_barrier()

    return body(hq, src2, dst2, coff)


def kernel(feat, edge_index, W1, b1, g1, e1, W2, b2, g2, e2, Wfc, bfc,
           Wna, Wsa, ca, ga, ea, Wn0, c0, g0, e0, Wn1, c1, gg1, ee1,
           Wf, bf, Wx, bx, Wo1, bo1, Wo2, bo2):
    n, d = feat.shape
    h_dim = W1.shape[1]
    e = edge_index.shape[1]
    c_dim = Wfc.shape[1]
    nh = Wf.shape[1] // h_dim
    nblk = n // _BLK
    fn = float(n)
    r = lambda x: x.reshape(1, -1)

    row = lambda bs: pl.BlockSpec(bs, lambda i: (i, 0))
    full = lambda shape: pl.BlockSpec(shape, lambda i: (0,) * len(shape))

    z1, st1 = pl.pallas_call(
        _a1_body,
        grid=(nblk,),
        in_specs=[row((_BLK, d)), full((d, h_dim)), full((1, h_dim))],
        out_specs=[row((_BLK, h_dim)), full((8, h_dim))],
        out_shape=[jax.ShapeDtypeStruct((n, h_dim), jnp.float32),
                   jax.ShapeDtypeStruct((8, h_dim), jnp.float32)],
    )(feat, W1, r(b1))

    z2, st2 = pl.pallas_call(
        functools.partial(_a2_body, n=fn),
        grid=(nblk,),
        in_specs=[row((_BLK, h_dim)), full((8, h_dim)), full((1, h_dim)),
                  full((1, h_dim)), full((h_dim, h_dim)), full((1, h_dim))],
        out_specs=[row((_BLK, h_dim)), full((8, h_dim))],
        out_shape=[jax.ShapeDtypeStruct((n, h_dim), jnp.float32),
                   jax.ShapeDtypeStruct((8, h_dim), jnp.float32)],
    )(z1, st1, r(g1), r(e1), W2, r(b2))

    h, hq, coff, clearf = pl.pallas_call(
        functools.partial(_a3_body, n=fn),
        grid=(nblk,),
        in_specs=[row((_BLK, h_dim)), full((8, h_dim)), full((1, h_dim)),
                  full((1, h_dim)), full((h_dim, c_dim)), full((1, c_dim))],
        out_specs=[row((_BLK, h_dim)),
                   pl.BlockSpec((8, _BLK, 32), lambda i: (0, i, 0)),
                   row((_BLK, 1)), row((_BLK, 1))],
        out_shape=[jax.ShapeDtypeStruct((n, h_dim), jnp.float32),
                   jax.ShapeDtypeStruct((8, n, 32), jnp.float32),
                   jax.ShapeDtypeStruct((n, 1), jnp.int32),
                   jax.ShapeDtypeStruct((n, 1), jnp.float32)],
    )(z2, st2, r(g2), r(e2), Wfc, r(bfc))

    e_pad = -(-e // (_NT * 1024)) * (_NT * 1024)
    pad = e_pad - e
    src = edge_index[0]
    dst = edge_index[1]
    if pad:
        src = jnp.concatenate([src, jnp.zeros((pad,), jnp.int32)])
        dst = jnp.concatenate(
            [dst, 2 * n + (jnp.arange(pad, dtype=jnp.int32) % 8)])
    src2 = src.reshape(e_pad // 128, 128)
    dst2 = dst.reshape(e_pad // 128, 128)
    sums_flat, cnt = _seg_sums_sc(hq.reshape(8 * n, 32), src2, dst2,
                                  coff.reshape(n), n, e_pad)
    sums = sums_flat.reshape(8, -1, 32)

    qblk = lambda off: pl.BlockSpec((8, _BLK, 32), lambda i, off=off: (0, i + off, 0))
    aall, a0, a1, stc = pl.pallas_call(
        _c1_body,
        grid=(nblk,),
        in_specs=[row((_BLK, h_dim)), qblk(0), qblk(nblk), row((_BLK, 1)),
                  full((h_dim, h_dim)), full((h_dim, h_dim)), full((1, h_dim)),
                  full((h_dim, h_dim)), full((1, h_dim)),
                  full((h_dim, h_dim)), full((1, h_dim))],
        out_specs=[row((_BLK, h_dim)), row((_BLK, h_dim)), row((_BLK, h_dim)),
                   full((8, h_dim))],
        out_shape=[jax.ShapeDtypeStruct((n, h_dim), jnp.float32),
                   jax.ShapeDtypeStruct((n, h_dim), jnp.float32),
                   jax.ShapeDtypeStruct((n, h_dim), jnp.float32),
                   jax.ShapeDtypeStruct((8, h_dim), jnp.float32)],
    )(h, sums, sums, cnt.reshape(n, 1), Wna, Wsa, r(ca), Wn0, r(c0), Wn1, r(c1))

    out = pl.pallas_call(
        functools.partial(_c2_body, n=fn, h_dim=h_dim, nh=nh),
        grid=(nblk,),
        in_specs=[row((_BLK, h_dim)), row((_BLK, h_dim)), row((_BLK, h_dim)),
                  full((8, h_dim)), row((_BLK, 1)), row((_BLK, h_dim)),
                  full((1, h_dim)), full((1, h_dim)), full((1, h_dim)),
                  full((1, h_dim)), full((1, h_dim)), full((1, h_dim)),
                  full((h_dim, nh * h_dim)), full((1, nh * h_dim)),
                  full((h_dim, nh * h_dim)), full((1, nh * h_dim)),
                  full((nh * h_dim, h_dim // 2)), full((1, h_dim // 2)),
                  full((h_dim // 2, c_dim)), full((1, c_dim))],
        out_specs=[row((_BLK, c_dim))],
        out_shape=[jax.ShapeDtypeStruct((n, c_dim), jnp.float32)],
    )(aall, a0, a1, stc, clearf, h, r(ga), r(ea), r(g0), r(e0), r(gg1), r(ee1),
      Wf, r(bf), Wx, r(bx), Wo1, r(bo1), Wo2, r(bo2))[0]

    return out


# speedup vs baseline: 1.0374x; 1.0033x over previous
import functools
import math

import jax
import jax.numpy as jnp
from jax import lax
from jax.experimental import pallas as pl
from jax.experimental.pallas import tpu as pltpu
from jax.experimental.pallas import tpu_sc as plsc

_EPS = 1e-5
_BLK = 1000


def _bn_from_stats(z, st_ref, row, g, e, n):
    m = st_ref[row : row + 1, :] / n
    v = st_ref[row + 1 : row + 2, :] / n - m * m
    return g * (z - m) * lax.rsqrt(v + _EPS) + e


def _stats_rows(arrs, h):
    rows = []
    for a in arrs:
        rows.append(jnp.sum(a, axis=0, keepdims=True))
        rows.append(jnp.sum(a * a, axis=0, keepdims=True))
    pad = 8 - len(rows)
    if pad:
        rows.append(jnp.zeros((pad, h), jnp.float32))
    return jnp.concatenate(rows, axis=0)


def _accum_stats(i, st_ref, st):
    @pl.when(i == 0)
    def _():
        st_ref[...] = st

    @pl.when(i > 0)
    def _():
        st_ref[...] = st_ref[...] + st


def _a1_body(feat_ref, w1_ref, b1_ref, z1_ref, st_ref):
    i = pl.program_id(0)
    z = jnp.dot(feat_ref[...], w1_ref[...], preferred_element_type=jnp.float32)
    z = z + b1_ref[...]
    z1_ref[...] = z
    _accum_stats(i, st_ref, _stats_rows([z], z.shape[1]))


def _a2_body(z1_ref, st1_ref, g1_ref, e1_ref, w2_ref, b2_ref, z2_ref, st_ref, *, n):
    i = pl.program_id(0)
    h1 = jnp.maximum(_bn_from_stats(z1_ref[...], st1_ref, 0, g1_ref[...], e1_ref[...], n), 0.0)
    z = jnp.dot(h1, w2_ref[...], preferred_element_type=jnp.float32) + b2_ref[...]
    z2_ref[...] = z
    _accum_stats(i, st_ref, _stats_rows([z], z.shape[1]))


def _a3_body(z2_ref, st2_ref, g2_ref, e2_ref, wfc_ref, bfc_ref,
             h_ref, hq_ref, coff_ref, clr_ref, *, n):
    h = jnp.maximum(_bn_from_stats(z2_ref[...], st2_ref, 0, g2_ref[...], e2_ref[...], n), 0.0)
    h_ref[...] = h
    for q in range(8):
        hq_ref[q] = h[:, q * 32 : (q + 1) * 32]
    logits = jnp.dot(h, wfc_ref[...], preferred_element_type=jnp.float32) + bfc_ref[...]
    mx = jnp.max(logits, axis=-1, keepdims=True)
    ex = jnp.exp(logits - mx)
    p = ex / jnp.sum(ex, axis=-1, keepdims=True)
    m1 = jnp.max(p, axis=-1, keepdims=True)
    ismax = p >= m1
    nmax = jnp.sum(ismax.astype(jnp.float32), axis=-1, keepdims=True)
    p2 = jnp.max(jnp.where(ismax, -1.0, p), axis=-1, keepdims=True)
    second = jnp.where(nmax > 1.5, m1, p2)
    unclear = (m1 - second) < 0.1
    coff_ref[...] = jnp.where(unclear, n, 0).astype(jnp.int32)
    clr_ref[...] = jnp.where(unclear, 0.0, 1.0)


def _c1_body(h_ref, smc_ref, smu_ref, cnt_ref, wna_ref, wsa_ref, ca_ref,
             wn0_ref, c0_ref, wn1_ref, c1_ref, aall_ref, a0_ref, a1_ref, st_ref):
    i = pl.program_id(0)
    sc = jnp.concatenate([smc_ref[q] for q in range(8)], axis=-1)
    su = jnp.concatenate([smu_ref[q] for q in range(8)], axis=-1)
    inv = 1.0 / jnp.maximum(cnt_ref[...], 1.0)
    h = h_ref[...]
    mall = (sc + su) * inv
    aall = jnp.dot(mall, wna_ref[...], preferred_element_type=jnp.float32) + ca_ref[...]
    aall = aall + jnp.dot(h, wsa_ref[...], preferred_element_type=jnp.float32)
    aall = jnp.maximum(aall, 0.0)
    a0 = jnp.maximum(jnp.dot(sc * inv, wn0_ref[...], preferred_element_type=jnp.float32) + c0_ref[...], 0.0)
    a1 = jnp.maximum(jnp.dot(su * inv, wn1_ref[...], preferred_element_type=jnp.float32) + c1_ref[...], 0.0)
    aall_ref[...] = aall
    a0_ref[...] = a0
    a1_ref[...] = a1
    _accum_stats(i, st_ref, _stats_rows([aall, a0, a1], aall.shape[1]))


def _c2_body(aall_ref, a0_ref, a1_ref, st_ref, clr_ref, h_ref,
             ga_ref, ea_ref, g0_ref, e0_ref, gg1_ref, ee1_ref,
             wf_ref, bf_ref, wx_ref, bx_ref, wo1_ref, bo1_ref, wo2_ref, bo2_ref,
             out_ref, *, n, h_dim, nh):
    hall = _bn_from_stats(aall_ref[...], st_ref, 0, ga_ref[...], ea_ref[...], n)
    h0 = _bn_from_stats(a0_ref[...], st_ref, 2, g0_ref[...], e0_ref[...], n)
    h1 = _bn_from_stats(a1_ref[...], st_ref, 4, gg1_ref[...], ee1_ref[...], n)
    clr = clr_ref[...]
    hgp = clr * h0 + (1.0 - clr) * h1
    bb = jnp.bfloat16
    wfb = wf_ref[...].astype(bb)
    fall = jnp.tanh(jnp.dot(hall.astype(bb), wfb, preferred_element_type=jnp.float32) + bf_ref[...])
    fgp = jnp.tanh(jnp.dot(hgp.astype(bb), wfb, preferred_element_type=jnp.float32) + bf_ref[...])
    x = jnp.tanh(jnp.dot(h_ref[...].astype(bb), wx_ref[...].astype(bb), preferred_element_type=jnp.float32) + bx_ref[...])
    scale = 1.0 / math.sqrt(float(h_dim))
    parts = []
    for hh in range(nh):
        lo = hh * h_dim
        fa = fall[:, lo : lo + h_dim]
        fg = fgp[:, lo : lo + h_dim]
        xh = x[:, lo : lo + h_dim]
        s0 = jnp.sum(fa * xh, axis=-1, keepdims=True) * scale
        s1 = jnp.sum(fg * xh, axis=-1, keepdims=True) * scale
        mx = jnp.maximum(s0, s1)
        e0 = jnp.exp(s0 - mx)
        e1 = jnp.exp(s1 - mx)
        den = e0 + e1
        parts.append((e0 / den) * fa + (e1 / den) * fg)
    agg = jnp.concatenate(parts, axis=-1)
    o = jnp.maximum(jnp.dot(agg, wo1_ref[...], preferred_element_type=jnp.float32) + bo1_ref[...], 0.0)
    out_ref[...] = jnp.dot(o, wo2_ref[...], preferred_element_type=jnp.float32) + bo2_ref[...]


_NT = 16


_FW = 32
_RING = 6


def _seg_sums_sc(hq, src2, dst2, coff, n, e_pad):
    v = n
    per_tile = e_pad // _NT
    idx_rows = per_tile // 128
    pchunks = per_tile // 5120
    acc_rows = 2 * v + 16
    cz = v // 10
    wb = (2 * v) // 10
    mesh = plsc.VectorSubcoreMesh(core_axis_name="c", subcore_axis_name="s")

    @functools.partial(
        pl.kernel,
        out_type=(
            jax.ShapeDtypeStruct((16 * v, _FW), jnp.float32),
            jax.ShapeDtypeStruct((v,), jnp.float32),
        ),
        mesh=mesh,
        scratch_types=(
            pltpu.VMEM((24, 128), jnp.int32),
            pltpu.VMEM((idx_rows, 128), jnp.int32),
            pltpu.VMEM((idx_rows, 128), jnp.int32),
            pltpu.VMEM((_RING * 128, _FW), jnp.float32),
            pltpu.VMEM((128,), jnp.float32),
            pltpu.VMEM((200,), jnp.float32),
            pltpu.VMEM_SHARED((v, _FW), jnp.float32),
            pltpu.VMEM_SHARED((acc_rows, _FW), jnp.float32),
            pltpu.VMEM_SHARED((v + 16,), jnp.float32),
            pltpu.VMEM_SHARED((v + 16,), jnp.int32),
            pltpu.SemaphoreType.DMA,
            pltpu.SemaphoreType.DMA,
            pltpu.SemaphoreType.DMA,
            pltpu.SemaphoreType.DMA,
        ),
        compiler_params=pltpu.CompilerParams(use_tc_tiling_on_sc=False),
    )
    def body(hq_h, src_h, dst_h, coff_h, out_h, cnt_h,
             ib, srca, sela, rowsb, onesb, zb, stable, acc, cacc, coffsp,
             semg, sems, semt, semn):
        c = lax.axis_index("c")
        s = lax.axis_index("s")
        lim = 2 * v + 15

        def fill16(i, _):
            onesb[pl.ds(i * 16, 16)] = jnp.ones((16,), jnp.float32)
            return 0

        lax.fori_loop(0, 8, fill16, 0)

        def zfill(i, _):
            zb[pl.ds(i * 16, 16)] = jnp.zeros((16,), jnp.float32)
            return 0

        lax.fori_loop(0, 200 // 16 + 1, zfill, 0)

        def zrows(i, _):
            for j in range(_FW // 16):
                rowsb[i, pl.ds(j * 16, 16)] = jnp.zeros((16,), jnp.float32)
            return 0

        lax.fori_loop(0, 384, zrows, 0)

        def zero_acc():
            base = s * 1248
            for j in range(4):
                pltpu.sync_copy(rowsb.at[pl.ds(0, 312)],
                                acc.at[pl.ds(base + j * 312, 312)])

            @pl.when(s == 0)
            def _():
                pltpu.sync_copy(rowsb.at[pl.ds(0, 48)],
                                acc.at[pl.ds(19968, 48)])

        def stage_table(p):
            @pl.when(s < 10)
            def _():
                pltpu.async_copy(
                    hq_h.at[pl.ds((c + 2 * p) * v + s * 1000, 1000)],
                    stable.at[pl.ds(s * 1000, 1000)], semt).wait()

        stage_table(0)

        @pl.when(s < 10)
        def _():
            pltpu.sync_copy(coff_h.at[pl.ds(s * 1000, 1000)],
                            coffsp.at[pl.ds(s * 1000, 1000)])

        zero_acc()

        @pl.when(jnp.logical_and(c == 0, s < 10))
        def _():
            for j in range(cz // 200):
                pltpu.sync_copy(zb.at[pl.ds(0, 200)],
                                cacc.at[pl.ds(s * cz + j * 200, 200)])

        plsc.subcore_barrier()

        for p in range(4):
            if p > 0:
                stage_table(p)
                lax.fori_loop(0, 384, zrows, 0)
                zero_acc()
                plsc.subcore_barrier()

            def edge_chunk(k, _, p=p):
                row0 = k * 40
                gds = {}
                sds = {}

                def buf(g):
                    return rowsb.at[pl.ds((g % _RING) * 128, 128)]

                if p == 0:
                    rb = s * idx_rows + k * 40
                    pltpu.sync_copy(src_h.at[pl.ds(rb, 40)],
                                    srca.at[pl.ds(row0, 40)])
                for g in range(_RING):
                    gds[g] = pltpu.async_copy(
                        stable.at[srca.at[row0 + g]], buf(g), semg)
                if p == 0:
                    for h2 in range(5):
                        o2 = 8 * h2
                        pltpu.sync_copy(dst_h.at[pl.ds(rb + o2, 8)],
                                        ib.at[pl.ds(8, 8)])
                        cps = [pltpu.async_copy(
                                   coffsp.at[srca.at[row0 + o2 + g]],
                                   ib.at[16 + g], semt)
                               for g in range(8)]
                        for cp in cps:
                            cp.wait()
                        for g in range(8):
                            def rloop(j, _, g=g, o2=o2):
                                sl = pl.ds(j * 16, 16)
                                dv = ib[8 + g, sl]
                                cv = ib[16 + g, sl]
                                sela[row0 + o2 + g, sl] = jnp.minimum(
                                    dv + cv, lim)
                                ib[16 + g, sl] = jnp.where(
                                    dv < v, dv,
                                    v + jnp.bitwise_and(dv, 7))
                                return 0

                            lax.fori_loop(0, 8, rloop, 0)

                        @pl.when(c == 0)
                        def _():
                            for g in range(8):
                                pltpu.async_copy(onesb,
                                                 cacc.at[ib.at[16 + g]],
                                                 semn, add=True)
                            for g in range(8):
                                pltpu.make_async_copy(onesb,
                                                      cacc.at[ib.at[16 + g]],
                                                      semn).wait()
                for g in range(40):
                    gds[g].wait()
                    sds[g] = pltpu.async_copy(
                        buf(g), acc.at[sela.at[row0 + g]], sems, add=True)
                    f = g + _RING - 2
                    if _RING <= f < 40:
                        sds[f - _RING].wait()
                        gds[f] = pltpu.async_copy(
                            stable.at[srca.at[row0 + f]], buf(f), semg)
                for g in range(40 - _RING, 40):
                    if g >= 0:
                        sds[g].wait()
                return 0

            lax.fori_loop(0, pchunks, edge_chunk, 0)
            plsc.subcore_barrier()

            if p == 0:

                @pl.when(jnp.logical_and(c == 0, s < 10))
                def _():
                    pltpu.sync_copy(cacc.at[pl.ds(s * cz, cz)],
                                    cnt_h.at[pl.ds(s * cz, cz)])

            q = c + 2 * p

            @pl.when(s < 10)
            def _():
                pltpu.sync_copy(acc.at[pl.ds(s * wb, wb)],
                                out_h.at[pl.ds(q * 2 * v + s * wb, wb)])

            plsc.subcore_barrier()

    return body(hq, src2, dst2, coff)


def kernel(feat, edge_index, W1, b1, g1, e1, W2, b2, g2, e2, Wfc, bfc,
           Wna, Wsa, ca, ga, ea, Wn0, c0, g0, e0, Wn1, c1, gg1, ee1,
           Wf, bf, Wx, bx, Wo1, bo1, Wo2, bo2):
    n, d = feat.shape
    h_dim = W1.shape[1]
    e = edge_index.shape[1]
    c_dim = Wfc.shape[1]
    nh = Wf.shape[1] // h_dim
    nblk = n // _BLK
    fn = float(n)
    r = lambda x: x.reshape(1, -1)

    row = lambda bs: pl.BlockSpec(bs, lambda i: (i, 0))
    full = lambda shape: pl.BlockSpec(shape, lambda i: (0,) * len(shape))

    z1, st1 = pl.pallas_call(
        _a1_body,
        grid=(nblk,),
        in_specs=[row((_BLK, d)), full((d, h_dim)), full((1, h_dim))],
        out_specs=[row((_BLK, h_dim)), full((8, h_dim))],
        out_shape=[jax.ShapeDtypeStruct((n, h_dim), jnp.float32),
                   jax.ShapeDtypeStruct((8, h_dim), jnp.float32)],
    )(feat, W1, r(b1))

    z2, st2 = pl.pallas_call(
        functools.partial(_a2_body, n=fn),
        grid=(nblk,),
        in_specs=[row((_BLK, h_dim)), full((8, h_dim)), full((1, h_dim)),
                  full((1, h_dim)), full((h_dim, h_dim)), full((1, h_dim))],
        out_specs=[row((_BLK, h_dim)), full((8, h_dim))],
        out_shape=[jax.ShapeDtypeStruct((n, h_dim), jnp.float32),
                   jax.ShapeDtypeStruct((8, h_dim), jnp.float32)],
    )(z1, st1, r(g1), r(e1), W2, r(b2))

    h, hq, coff, clearf = pl.pallas_call(
        functools.partial(_a3_body, n=fn),
        grid=(nblk,),
        in_specs=[row((_BLK, h_dim)), full((8, h_dim)), full((1, h_dim)),
                  full((1, h_dim)), full((h_dim, c_dim)), full((1, c_dim))],
        out_specs=[row((_BLK, h_dim)),
                   pl.BlockSpec((8, _BLK, 32), lambda i: (0, i, 0)),
                   row((_BLK, 1)), row((_BLK, 1))],
        out_shape=[jax.ShapeDtypeStruct((n, h_dim), jnp.float32),
                   jax.ShapeDtypeStruct((8, n, 32), jnp.float32),
                   jax.ShapeDtypeStruct((n, 1), jnp.int32),
                   jax.ShapeDtypeStruct((n, 1), jnp.float32)],
    )(z2, st2, r(g2), r(e2), Wfc, r(bfc))

    e_pad = -(-e // (_NT * 1024)) * (_NT * 1024)
    pad = e_pad - e
    src = edge_index[0]
    dst = edge_index[1]
    if pad:
        src = jnp.concatenate([src, jnp.zeros((pad,), jnp.int32)])
        dst = jnp.concatenate(
            [dst, 2 * n + (jnp.arange(pad, dtype=jnp.int32) % 8)])
    src2 = src.reshape(e_pad // 128, 128)
    dst2 = dst.reshape(e_pad // 128, 128)
    sums_flat, cnt = _seg_sums_sc(hq.reshape(8 * n, 32), src2, dst2,
                                  coff.reshape(n), n, e_pad)
    sums = sums_flat.reshape(8, -1, 32)

    qblk = lambda off: pl.BlockSpec((8, _BLK, 32), lambda i, off=off: (0, i + off, 0))
    aall, a0, a1, stc = pl.pallas_call(
        _c1_body,
        grid=(nblk,),
        in_specs=[row((_BLK, h_dim)), qblk(0), qblk(nblk), row((_BLK, 1)),
                  full((h_dim, h_dim)), full((h_dim, h_dim)), full((1, h_dim)),
                  full((h_dim, h_dim)), full((1, h_dim)),
                  full((h_dim, h_dim)), full((1, h_dim))],
        out_specs=[row((_BLK, h_dim)), row((_BLK, h_dim)), row((_BLK, h_dim)),
                   full((8, h_dim))],
        out_shape=[jax.ShapeDtypeStruct((n, h_dim), jnp.float32),
                   jax.ShapeDtypeStruct((n, h_dim), jnp.float32),
                   jax.ShapeDtypeStruct((n, h_dim), jnp.float32),
                   jax.ShapeDtypeStruct((8, h_dim), jnp.float32)],
    )(h, sums, sums, cnt.reshape(n, 1), Wna, Wsa, r(ca), Wn0, r(c0), Wn1, r(c1))

    out = pl.pallas_call(
        functools.partial(_c2_body, n=fn, h_dim=h_dim, nh=nh),
        grid=(nblk,),
        in_specs=[row((_BLK, h_dim)), row((_BLK, h_dim)), row((_BLK, h_dim)),
                  full((8, h_dim)), row((_BLK, 1)), row((_BLK, h_dim)),
                  full((1, h_dim)), full((1, h_dim)), full((1, h_dim)),
                  full((1, h_dim)), full((1, h_dim)), full((1, h_dim)),
                  full((h_dim, nh * h_dim)), full((1, nh * h_dim)),
                  full((h_dim, nh * h_dim)), full((1, nh * h_dim)),
                  full((nh * h_dim, h_dim // 2)), full((1, h_dim // 2)),
                  full((h_dim // 2, c_dim)), full((1, c_dim))],
        out_specs=[row((_BLK, c_dim))],
        out_shape=[jax.ShapeDtypeStruct((n, c_dim), jnp.float32)],
    )(aall, a0, a1, stc, clearf, h, r(ga), r(ea), r(g0), r(e0), r(gg1), r(ee1),
      Wf, r(bf), Wx, r(bx), Wo1, r(bo1), Wo2, r(bo2))[0]

    return out
